# baseline (speedup 1.0000x reference)
import functools

import jax
import jax.numpy as jnp
from jax import lax
from jax.experimental import pallas as pl
from jax.experimental.pallas import tpu as pltpu
from jax.experimental.pallas import tpu_sc as plsc

R = 128
V = 100000
NW = 32
ROWS_PER_W = R // NW
CHUNK = 20000
NCHUNK = V // CHUNK
VECS = CHUNK // 16
UNROLL = 10

_mesh = plsc.VectorSubcoreMesh(core_axis_name="c", subcore_axis_name="s")


@functools.partial(
    pl.kernel,
    mesh=_mesh,
    compiler_params=pltpu.CompilerParams(needs_layout_passes=False),
    out_type=jax.ShapeDtypeStruct((NW, 16), jnp.int32),
    scratch_types=[
        pltpu.VMEM((CHUNK,), jnp.float32),
        pltpu.VMEM((CHUNK,), jnp.float32),
        pltpu.VMEM((16,), jnp.int32),
        pltpu.SemaphoreType.DMA,
        pltpu.SemaphoreType.DMA,
    ],
)
def _argmax_kernel(x_hbm, out_hbm, buf0, buf1, obuf, sem0, sem1):
    wid = lax.axis_index("s") * 2 + lax.axis_index("c")
    base_elem = wid * (ROWS_PER_W * V)
    iota = lax.iota(jnp.int32, 16)
    bufs = (buf0, buf1)
    sems = (sem0, sem1)

    def src(i):
        return x_hbm.at[pl.ds(base_elem + i * CHUNK, CHUNK)]

    nflat = ROWS_PER_W * NCHUNK
    copies = [None, None]
    copies[0] = pltpu.async_copy(src(0), bufs[0], sems[0])

    result = jnp.zeros((16,), jnp.int32)
    neg_inf = jnp.full((16,), -jnp.inf, jnp.float32)

    vm = neg_inf
    vi = jnp.zeros((16,), jnp.int32)
    for i in range(nflat):
        b = i % 2
        if i + 1 < nflat:
            nb = (i + 1) % 2
            copies[nb] = pltpu.async_copy(src(i + 1), bufs[nb], sems[nb])
        copies[b].wait()
        buf = bufs[b]
        c = i % NCHUNK
        r = i // NCHUNK
        base_cols = [iota + (c * CHUNK + u * 16) for u in range(UNROLL)]

        def body(t, carry, _buf=buf, _base_cols=base_cols):
            bvm, bvi = carry
            off = t * (UNROLL * 16)
            for u in range(UNROLL):
                v = _buf[pl.ds(off + u * 16, 16)]
                col = _base_cols[u] + off
                pred = v > bvm
                bvm = jnp.where(pred, v, bvm)
                bvi = jnp.where(pred, col, bvi)
            return bvm, bvi

        vm, vi = lax.fori_loop(0, VECS // UNROLL, body, (vm, vi))

        if c == NCHUNK - 1:
            sk = plsc.sort_key_val(vm, vi, descending=True)[0]
            m = sk[0]
            cand = jnp.where(vm == m, vi, jnp.int32(V))
            ck = plsc.sort_key_val(cand, cand)[0]
            ans = ck[0]
            result = jnp.where(iota == r, ans, result)
            vm = neg_inf
            vi = jnp.zeros((16,), jnp.int32)

    obuf[...] = result
    pltpu.sync_copy(obuf, out_hbm.at[wid])


def kernel(m_logits):
    flat = m_logits.reshape(-1)
    out = _argmax_kernel(flat)
    return out[:, :ROWS_PER_W].reshape(R, 1)


# speedup vs baseline: 1.0830x; 1.0830x over previous
import functools

import jax
import jax.numpy as jnp
from jax import lax
from jax.experimental import pallas as pl
from jax.experimental.pallas import tpu as pltpu
from jax.experimental.pallas import tpu_sc as plsc

R = 128
V = 100000
NW = 32
ROWS_PER_W = R // NW
CHUNK = 20000
NCHUNK = V // CHUNK
VECS = CHUNK // 16
UNROLL = 10
ITERS = VECS // UNROLL
NBUF = 3

_mesh = plsc.VectorSubcoreMesh(core_axis_name="c", subcore_axis_name="s")


@functools.partial(
    pl.kernel,
    mesh=_mesh,
    compiler_params=pltpu.CompilerParams(needs_layout_passes=False),
    out_type=jax.ShapeDtypeStruct((NW, 16), jnp.int32),
    scratch_types=[
        pltpu.VMEM((CHUNK,), jnp.float32),
        pltpu.VMEM((CHUNK,), jnp.float32),
        pltpu.VMEM((CHUNK,), jnp.float32),
        pltpu.VMEM((16,), jnp.int32),
        pltpu.SemaphoreType.DMA,
        pltpu.SemaphoreType.DMA,
        pltpu.SemaphoreType.DMA,
    ],
)
def _argmax_kernel(x_hbm, out_hbm, buf0, buf1, buf2, obuf, sem0, sem1, sem2):
    bufs = (buf0, buf1, buf2)
    wid = lax.axis_index("s") * 2 + lax.axis_index("c")
    base_elem = wid * (ROWS_PER_W * V)
    iota = lax.iota(jnp.int32, 16)
    sems = (sem0, sem1, sem2)

    def src(i):
        return x_hbm.at[pl.ds(base_elem + i * CHUNK, CHUNK)]

    nflat = ROWS_PER_W * NCHUNK
    copies = [None] * NBUF
    for i in range(NBUF - 1):
        copies[i] = pltpu.async_copy(src(i), bufs[i], sems[i])

    result = jnp.zeros((16,), jnp.int32)
    neg_inf = jnp.full((16,), -jnp.inf, jnp.float32)
    slot_off = [iota + 16 * u for u in range(UNROLL)]

    vms = [neg_inf] * UNROLL
    vts = [jnp.zeros((16,), jnp.int32)] * UNROLL
    for i in range(nflat):
        b = i % NBUF
        if i + NBUF - 1 < nflat:
            nb = (i + NBUF - 1) % NBUF
            copies[nb] = pltpu.async_copy(src(i + NBUF - 1), bufs[nb], sems[nb])
        copies[b].wait()
        c = i % NCHUNK
        r = i // NCHUNK

        def body(t, carry, _buf=bufs[b], _c=c):
            acc = list(carry)
            tv = jnp.broadcast_to(t + _c * ITERS, (16,))
            off = t * (UNROLL * 16)
            for u in range(UNROLL):
                v = _buf[pl.ds(off + u * 16, 16)]
                pred = v > acc[u]
                acc[u] = jnp.where(pred, v, acc[u])
                acc[UNROLL + u] = jnp.where(pred, tv, acc[UNROLL + u])
            return tuple(acc)

        out_carry = lax.fori_loop(0, ITERS, body, tuple(vms) + tuple(vts))
        vms = list(out_carry[:UNROLL])
        vts = list(out_carry[UNROLL:])

        if c == NCHUNK - 1:
            cols = [vts[u] * (UNROLL * 16) + slot_off[u] for u in range(UNROLL)]
            mv, mc = vms[0], cols[0]
            for u in range(1, UNROLL):
                pred = (mv > vms[u]) | ((mv == vms[u]) & (mc < cols[u]))
                mv = jnp.where(pred, mv, vms[u])
                mc = jnp.where(pred, mc, cols[u])
            sk = plsc.sort_key_val(mv, mc, descending=True)[0]
            m = sk[0]
            cand = jnp.where(mv == m, mc, jnp.int32(V))
            ans = plsc.sort_key_val(cand, cand)[0][0]
            result = jnp.where(iota == r, ans, result)
            vms = [neg_inf] * UNROLL
            vts = [jnp.zeros((16,), jnp.int32)] * UNROLL

    obuf[...] = result
    pltpu.sync_copy(obuf, out_hbm.at[wid])


def kernel(m_logits):
    flat = m_logits.reshape(-1)
    out = _argmax_kernel(flat)
    return out[:, :ROWS_PER_W].reshape(R, 1)


# speedup vs baseline: 1.7179x; 1.5862x over previous
import functools

import jax
import jax.numpy as jnp
from jax import lax
from jax.experimental import pallas as pl
from jax.experimental.pallas import tpu as pltpu
from jax.experimental.pallas import tpu_sc as plsc

R = 128
V = 100000
VA = 99968
NW = 32
HALF = 50048
CW = 2048
NFULL = 24
RAG0 = HALF - NFULL * CW
RAG1 = (VA - HALF) - NFULL * CW
UNROLL = 8
ITERS = CW // (16 * UNROLL)
NEG_INF = float("-inf")

_mesh = plsc.VectorSubcoreMesh(core_axis_name="c", subcore_axis_name="s")


@functools.partial(
    pl.kernel,
    mesh=_mesh,
    compiler_params=pltpu.CompilerParams(needs_layout_passes=False),
    out_type=(
        jax.ShapeDtypeStruct((NW, 16), jnp.float32),
        jax.ShapeDtypeStruct((NW, 16), jnp.int32),
    ),
    scratch_types=[
        pltpu.VMEM((8, CW), jnp.float32),
        pltpu.VMEM((8, CW), jnp.float32),
        pltpu.VMEM((8, CW), jnp.float32),
        pltpu.VMEM((8, RAG0), jnp.float32),
        pltpu.VMEM((8, 128), jnp.float32),
        pltpu.VMEM((16,), jnp.float32),
        pltpu.VMEM((16,), jnp.int32),
        pltpu.SemaphoreType.DMA,
        pltpu.SemaphoreType.DMA,
        pltpu.SemaphoreType.DMA,
        pltpu.SemaphoreType.DMA,
        pltpu.SemaphoreType.DMA,
    ],
)
def _argmax_kernel(x_hbm, tail_hbm, outv_hbm, outc_hbm, b0, b1, b2, brag,
                   btail, obufv, obufc, s0, s1, s2, srag, stail):
    sub = lax.axis_index("s")
    core = lax.axis_index("c")
    wid = sub * 2 + core
    g = sub
    h = core
    row0 = g * 8
    hstart = h * HALF
    iota = lax.iota(jnp.int32, 16)
    neg_inf = jnp.full((16,), NEG_INF, jnp.float32)
    zero_i = jnp.zeros((16,), jnp.int32)
    slot_off = [iota + 16 * u for u in range(UNROLL)]
    bufs = (b0, b1, b2)
    sems = (s0, s1, s2)

    def blk_src(blk, width):
        c0 = pl.multiple_of(hstart + blk * CW, 128)
        return x_hbm.at[pl.ds(row0, 8), pl.ds(c0, width)]

    for b in range(3):
        pltpu.async_copy(blk_src(b, CW), bufs[b], sems[b])
    pltpu.async_copy(tail_hbm.at[pl.ds(row0, 8), pl.ds(0, 128)], btail, stail)

    @pl.when(h == 0)
    def _():
        pltpu.async_copy(blk_src(NFULL, RAG0), brag, srag)

    @pl.when(h == 1)
    def _():
        pltpu.async_copy(
            blk_src(NFULL, RAG1),
            brag.at[pl.ds(0, 8), pl.ds(0, RAG1)], srag)

    def scan_block(buf, r, c0v, niters, run_v, run_c):
        init = tuple([neg_inf] * UNROLL + [zero_i] * UNROLL)

        def body(t, carry):
            acc = list(carry)
            tv = jnp.broadcast_to(t, (16,))
            off = t * (UNROLL * 16)
            for u in range(UNROLL):
                v = buf[r, pl.ds(off + u * 16, 16)]
                pred = v > acc[u]
                acc[u] = jnp.where(pred, v, acc[u])
                acc[UNROLL + u] = jnp.where(pred, tv, acc[UNROLL + u])
            return tuple(acc)

        acc = list(lax.fori_loop(0, niters, body, init))
        vs = acc[:UNROLL]
        cs = [(acc[UNROLL + u] << 7) + slot_off[u] + c0v for u in range(UNROLL)]
        n = UNROLL
        while n > 1:
            n //= 2
            for k in range(n):
                a, b = k, k + n
                pred = (vs[a] > vs[b]) | ((vs[a] == vs[b]) & (cs[a] < cs[b]))
                vs[k] = jnp.where(pred, vs[a], vs[b])
                cs[k] = jnp.where(pred, cs[a], cs[b])
        keep = run_v >= vs[0]
        return jnp.where(keep, run_v, vs[0]), jnp.where(keep, run_c, cs[0])

    def outer(j, carry):
        rv = list(carry[:8])
        rc = list(carry[8:])
        for b in range(3):
            blk = j * 3 + b
            pltpu.make_async_copy(blk_src(0, CW), bufs[b], sems[b]).wait()
            c0v = jnp.broadcast_to(blk * CW, (16,))
            for r in range(8):
                rv[r], rc[r] = scan_block(bufs[b], r, c0v, ITERS, rv[r], rc[r])

            @pl.when(j < (NFULL // 3) - 1)
            def _():
                pltpu.async_copy(blk_src(blk + 3, CW), bufs[b], sems[b])

        return tuple(rv) + tuple(rc)

    carry = lax.fori_loop(
        0, NFULL // 3, outer, tuple([neg_inf] * 8 + [zero_i] * 8))
    run_v = list(carry[:8])
    run_c = list(carry[8:])

    @pl.when(h == 0)
    def _():
        pltpu.make_async_copy(blk_src(0, RAG0), brag, srag).wait()

    @pl.when(h == 1)
    def _():
        pltpu.make_async_copy(
            blk_src(0, RAG1),
            brag.at[pl.ds(0, 8), pl.ds(0, RAG1)], srag).wait()
    rag_iters = (RAG0 // (16 * UNROLL)) - h
    c0v = jnp.broadcast_to(NFULL * CW, (16,))
    for r in range(8):
        run_v[r], run_c[r] = scan_block(
            brag, r, c0v, rag_iters, run_v[r], run_c[r])

    pltpu.make_async_copy(blk_src(0, 128), btail, stail).wait()
    c0v = jnp.broadcast_to(VA - hstart, (16,))
    for r in range(8):
        run_v[r], run_c[r] = scan_block(
            btail, r, c0v, 1, run_v[r], run_c[r])

    res_v = jnp.zeros((16,), jnp.float32)
    res_c = zero_i
    hstart_v = jnp.broadcast_to(hstart, (16,))
    for r in range(8):
        m = plsc.sort_key_val(run_v[r], run_c[r], descending=True)[0][0]
        cand = jnp.where(run_v[r] == m, run_c[r], jnp.int32(V))
        ans = plsc.sort_key_val(cand, cand)[0][0]
        res_v = jnp.where(iota == r, m, res_v)
        res_c = jnp.where(iota == r, ans, res_c)
    obufv[...] = res_v
    obufc[...] = res_c + hstart_v
    pltpu.sync_copy(obufv, outv_hbm.at[wid])
    pltpu.sync_copy(obufc, outc_hbm.at[wid])


def kernel(m_logits):
    tail = jnp.pad(m_logits[:, VA:], ((0, 0), (0, 128 - (V - VA))),
                   constant_values=NEG_INF)
    outv, outc = _argmax_kernel(m_logits, tail)
    v = outv.reshape(16, 2, 16)[:, :, :8]
    c = outc.reshape(16, 2, 16)[:, :, :8]
    pick = (v[:, 1] > v[:, 0]) | ((v[:, 1] == v[:, 0]) & (c[:, 1] < c[:, 0]))
    col = jnp.where(pick, c[:, 1], c[:, 0])
    return col.reshape(R, 1).astype(jnp.int32)


# speedup vs baseline: 3.7371x; 2.1754x over previous
import functools

import jax
import jax.numpy as jnp
from jax import lax
from jax.experimental import pallas as pl
from jax.experimental.pallas import tpu as pltpu
from jax.experimental.pallas import tpu_sc as plsc

R = 128
V = 100000
NW = 32
STRIDE = 3120
WINDOW = 3280
CB = 328
NBLK = WINDOW // CB
GROUPS = R // 16
NEG_INF = float("-inf")

_mesh = plsc.VectorSubcoreMesh(core_axis_name="c", subcore_axis_name="s")


@functools.partial(
    pl.kernel,
    mesh=_mesh,
    compiler_params=pltpu.CompilerParams(needs_layout_passes=False),
    out_type=(
        jax.ShapeDtypeStruct((NW, GROUPS, 16), jnp.float32),
        jax.ShapeDtypeStruct((NW, GROUPS, 16), jnp.int32),
    ),
    scratch_types=[
        pltpu.VMEM((CB, R), jnp.float32),
        pltpu.VMEM((CB, R), jnp.float32),
        pltpu.VMEM((CB, R), jnp.float32),
        pltpu.VMEM((GROUPS, 16), jnp.float32),
        pltpu.VMEM((GROUPS, 16), jnp.int32),
        pltpu.SemaphoreType.DMA,
        pltpu.SemaphoreType.DMA,
        pltpu.SemaphoreType.DMA,
    ],
)
def _argmax_kernel(xt_hbm, outv_hbm, outc_hbm, b0, b1, b2, obufv, obufc,
                   s0, s1, s2):
    wid = lax.axis_index("s") * 2 + lax.axis_index("c")
    wstart = wid * STRIDE
    bufs = (b0, b1, b2)
    sems = (s0, s1, s2)

    def blk_src(blk):
        c0 = pl.multiple_of(wstart + blk * CB, 8)
        return xt_hbm.at[pl.ds(c0, CB), pl.ds(0, R)]

    for b in range(3):
        pltpu.async_copy(blk_src(b), bufs[b], sems[b])

    neg_inf = jnp.full((16,), NEG_INF, jnp.float32)
    zero_i = jnp.zeros((16,), jnp.int32)

    def scan_block(buf, cbase, accs):
        def body(c, carry):
            acc = list(carry)
            cv = jnp.broadcast_to(cbase + c, (16,))
            for u in range(GROUPS):
                v = buf[c, pl.ds(u * 16, 16)]
                pred = v > acc[u]
                acc[u] = jnp.where(pred, v, acc[u])
                acc[GROUPS + u] = jnp.where(pred, cv, acc[GROUPS + u])
            return tuple(acc)

        return lax.fori_loop(0, CB, body, accs)

    def outer(j, carry):
        acc = carry
        for b in range(3):
            blk = j * 3 + b
            pltpu.make_async_copy(blk_src(0), bufs[b], sems[b]).wait()
            acc = scan_block(bufs[b], wstart + blk * CB, acc)

            @pl.when(blk + 3 < NBLK)
            def _():
                pltpu.async_copy(blk_src(blk + 3), bufs[b], sems[b])

        return acc

    init = tuple([neg_inf] * GROUPS + [zero_i] * GROUPS)
    acc = lax.fori_loop(0, (NBLK // 3), outer, init)

    pltpu.make_async_copy(blk_src(0), bufs[0], sems[0]).wait()
    acc = scan_block(bufs[0], wstart + (NBLK - 1) * CB, acc)

    for u in range(GROUPS):
        obufv[u, pl.ds(0, 16)] = acc[u]
        obufc[u, pl.ds(0, 16)] = acc[GROUPS + u]
    pltpu.sync_copy(obufv, outv_hbm.at[wid])
    pltpu.sync_copy(obufc, outc_hbm.at[wid])


def kernel(m_logits):
    outv, outc = _argmax_kernel(m_logits.T)
    vals = outv.reshape(NW, R)
    cols = outc.reshape(NW, R)
    m = vals.max(axis=0)
    cand = jnp.where(vals == m[None, :], cols, jnp.int32(V))
    return cand.min(axis=0).reshape(R, 1).astype(jnp.int32)
